# baseline (speedup 1.0000x reference)
import jax
import jax.numpy as jnp
from jax.experimental import pallas as pl
from jax.experimental.pallas import tpu as pltpu


def _act_body(rgb_ref, opa_ref, scl_ref, rgb_out, opa_out, scl_out):
    rgb_out[...] = jax.nn.sigmoid(rgb_ref[...])
    opa_out[...] = jax.nn.sigmoid(opa_ref[...])
    scl_out[...] = jnp.exp(scl_ref[...])


def kernel(xyz_raw, rgb_raw, opacity_raw, scale_raw):
    n = rgb_raw.shape[0]
    rows = n // 128
    rgb2 = rgb_raw.reshape(rows, 384)
    opa2 = opacity_raw.reshape(rows, 128)
    scl2 = scale_raw.reshape(rows, 128)

    R = 512
    grid = (pl.cdiv(rows, R),)
    rgb_o, opa_o, scl_o = pl.pallas_call(
        _act_body,
        grid=grid,
        in_specs=[
            pl.BlockSpec((R, 384), lambda i: (i, 0)),
            pl.BlockSpec((R, 128), lambda i: (i, 0)),
            pl.BlockSpec((R, 128), lambda i: (i, 0)),
        ],
        out_specs=[
            pl.BlockSpec((R, 384), lambda i: (i, 0)),
            pl.BlockSpec((R, 128), lambda i: (i, 0)),
            pl.BlockSpec((R, 128), lambda i: (i, 0)),
        ],
        out_shape=[
            jax.ShapeDtypeStruct((rows, 384), jnp.float32),
            jax.ShapeDtypeStruct((rows, 128), jnp.float32),
            jax.ShapeDtypeStruct((rows, 128), jnp.float32),
        ],
        compiler_params=pltpu.CompilerParams(
            dimension_semantics=("arbitrary",),
        ),
    )(rgb2, opa2, scl2)

    return (
        xyz_raw,
        rgb_o.reshape(n, 3),
        opa_o.reshape(n, 1),
        scl_o.reshape(n, 1),
    )


# speedup vs baseline: 30.7635x; 30.7635x over previous
import jax
import jax.numpy as jnp
from jax.experimental import pallas as pl
from jax.experimental.pallas import tpu as pltpu


def _act_body(rgb_ref, opa_ref, scl_ref, rgb_out, opa_out, scl_out):
    rgb_out[...] = jax.nn.sigmoid(rgb_ref[...])
    opa_out[...] = jax.nn.sigmoid(opa_ref[...])
    scl_out[...] = jnp.exp(scl_ref[...])


def kernel(xyz_raw, rgb_raw, opacity_raw, scale_raw):
    n = rgb_raw.shape[0]
    rows = n // 128
    rgbT = rgb_raw.T
    opa2 = opacity_raw.reshape(rows, 128)
    scl2 = scale_raw.reshape(rows, 128)

    C = 131072
    RO = C // 128
    grid = (pl.cdiv(n, C),)
    rgb_o, opa_o, scl_o = pl.pallas_call(
        _act_body,
        grid=grid,
        in_specs=[
            pl.BlockSpec((3, C), lambda i: (0, i)),
            pl.BlockSpec((RO, 128), lambda i: (i, 0)),
            pl.BlockSpec((RO, 128), lambda i: (i, 0)),
        ],
        out_specs=[
            pl.BlockSpec((3, C), lambda i: (0, i)),
            pl.BlockSpec((RO, 128), lambda i: (i, 0)),
            pl.BlockSpec((RO, 128), lambda i: (i, 0)),
        ],
        out_shape=[
            jax.ShapeDtypeStruct((3, n), jnp.float32),
            jax.ShapeDtypeStruct((rows, 128), jnp.float32),
            jax.ShapeDtypeStruct((rows, 128), jnp.float32),
        ],
        compiler_params=pltpu.CompilerParams(
            dimension_semantics=("arbitrary",),
        ),
    )(rgbT, opa2, scl2)

    return (
        xyz_raw,
        rgb_o.T,
        opa_o.reshape(n, 1),
        scl_o.reshape(n, 1),
    )


# speedup vs baseline: 103.1224x; 3.3521x over previous
import jax
import jax.numpy as jnp
from jax.experimental import pallas as pl
from jax.experimental.pallas import tpu as pltpu


def _act_body(rgb_ref, rgb_out):
    rgb_out[...] = jax.nn.sigmoid(rgb_ref[...])


def kernel(xyz_raw, rgb_raw, opacity_raw, scale_raw):
    n = rgb_raw.shape[0]
    rgbT = rgb_raw.T

    C = 131072
    grid = (pl.cdiv(n, C),)
    rgb_o = pl.pallas_call(
        _act_body,
        grid=grid,
        in_specs=[pl.BlockSpec((3, C), lambda i: (0, i))],
        out_specs=pl.BlockSpec((3, C), lambda i: (0, i)),
        out_shape=jax.ShapeDtypeStruct((3, n), jnp.float32),
        compiler_params=pltpu.CompilerParams(
            dimension_semantics=("arbitrary",),
        ),
    )(rgbT)

    return (
        xyz_raw,
        rgb_o.T,
        jax.nn.sigmoid(opacity_raw),
        jnp.exp(scale_raw),
    )
